# Initial kernel scaffold; baseline (speedup 1.0000x reference)
import jax
import jax.numpy as jnp
from jax.experimental import pallas as pl


def kernel(x, edge_index, edge_attr, node_type, edge_type, lin_W, lin_b, e0_W, e0_b, e1_W, e1_b, emb, ln_g, ln_b, fc_W, fc_b):
    raise NotImplementedError("write your pallas kernel here")


# speedup vs baseline: 3.0852x; 3.0852x over previous
import functools

import jax
import jax.numpy as jnp
from jax import lax
from jax.experimental import pallas as pl
from jax.experimental.pallas import tpu as pltpu
from jax.experimental.pallas import tpu_sc as plsc

N = 10000
E = 320000
D = 128
ED = 16
NT = 2
ET = 2
L = 2

NW = 32
G = 128
EPW = E // NW
NB = (EPW + G - 1) // G
EPW_P = NB * G
Q = 50
NST = N // Q
NP = 10240
ROWS_PER_TILE = NP // 16


def _edge_coeff_body(ea_ref, et_ref, wc_ref, off_ref, out_ref):
    ea = ea_ref[...]
    et = et_ref[...]
    raw = jnp.dot(ea, wc_ref[...], preferred_element_type=jnp.float32)
    off = jnp.where(et == 0, off_ref[0:1, :], off_ref[1:2, :])
    raw = raw + off
    d = ea[:, ED - 2:ED - 1]
    p = ea[:, ED - 1:ED]
    sign = d * 2.0 - 1.0
    speed_local = p * jnp.where(d > 0.0, d, 1.0)
    m0 = (et == 0).astype(jnp.float32)
    m1 = (et == 1).astype(jnp.float32)

    def softplus(v):
        return jnp.maximum(v, 0.0) + jnp.log1p(jnp.exp(-jnp.abs(v)))

    cols = []
    for l in range(L):
        r0 = raw[:, 3 * l + 0:3 * l + 1]
        r1a = raw[:, 3 * l + 1:3 * l + 2]
        r1b = raw[:, 3 * l + 2:3 * l + 3]
        gain = m0 * softplus(r0) + m1 * softplus(r1a) * speed_local
        bias = m1 * r1b * speed_local
        cols.append(sign * gain)
        cols.append(sign * bias)
    out_ref[...] = jnp.concatenate(cols, axis=1)


_EB = 8000


def _edge_coeffs(edge_attr, edge_type2, wc, off):
    grid = E // _EB
    return pl.pallas_call(
        _edge_coeff_body,
        grid=(grid,),
        in_specs=[
            pl.BlockSpec((_EB, ED), lambda i: (i, 0)),
            pl.BlockSpec((_EB, 1), lambda i: (i, 0)),
            pl.BlockSpec((ED, 8), lambda i: (0, 0)),
            pl.BlockSpec((ET, 8), lambda i: (0, 0)),
        ],
        out_specs=pl.BlockSpec((_EB, 2 * L), lambda i: (i, 0)),
        out_shape=jax.ShapeDtypeStruct((E, 2 * L), jnp.float32),
    )(edge_attr, edge_type2, wc, off)


def _sc_spmm_body(x_hbm, meta_hbm, out_hbm, srcb, dstb, strb, colb, ab, cb,
                  rows, strows, acc, sem, msem):
    cid = lax.axis_index("c")
    sid = lax.axis_index("s")
    wid = cid * 16 + sid

    zero16 = jnp.zeros((16,), jnp.float32)

    @pl.loop(0, G)
    def _zero_rows(i):
        for j in range(D // 16):
            rows[i, pl.ds(j * 16, 16)] = zero16
            strows[i, pl.ds(j * 16, 16)] = zero16

    for k in range(ROWS_PER_TILE // G):
        pltpu.sync_copy(rows, acc.at[pl.ds(sid * ROWS_PER_TILE + k * G, G)])
    plsc.subcore_barrier()

    lane = lax.iota(jnp.int32, 16)

    bufs = [srcb, dstb, strb, colb, ab, cb]
    mbase = wid * (NB * 6)

    def _fetch(b, p):
        for k, buf in enumerate(bufs):
            pltpu.async_copy(meta_hbm.at[mbase + b * 6 + k], buf.at[p], msem)

    def _drain(b, p):
        for k, buf in enumerate(bufs):
            pltpu.make_async_copy(meta_hbm.at[mbase + b * 6 + k], buf.at[p],
                                  msem).wait()

    _fetch(0, 0)
    _drain(0, 0)

    def _do_batch(b, p):
        @pl.when(b + 1 < NB)
        def _pref():
            _fetch(b + 1, 1 - p)

        pltpu.async_copy(x_hbm.at[srcb.at[p]], rows, sem).wait()

        @pl.loop(0, G // 16)
        def _grp(gg):
            sl16 = pl.ds(gg * 16, 16)
            a16 = plsc.bitcast(ab[p, sl16], jnp.float32)
            c16 = plsc.bitcast(cb[p, sl16], jnp.float32)
            col16 = colb[p, sl16]
            g16 = lane + gg * 16
            plsc.store_scatter(strows, [g16, col16], a16)
            plsc.store_scatter(strows, [g16, col16 + 1], c16)
            for j in range(16):
                g = gg * 16 + j
                av = jnp.full((16,), a16[j], jnp.float32)
                for dcol in range(D // 16):
                    sl = pl.ds(dcol * 16, 16)
                    rows[g, sl] = rows[g, sl] * av

        pltpu.sync_copy(rows, acc.at[dstb.at[p]], add=True)
        pltpu.sync_copy(strows, acc.at[strb.at[p]], add=True)

        @pl.loop(0, G // 16)
        def _unzero(gg):
            col16 = colb[p, pl.ds(gg * 16, 16)]
            g16 = lane + gg * 16
            plsc.store_scatter(strows, [g16, col16], zero16)
            plsc.store_scatter(strows, [g16, col16 + 1], zero16)

        @pl.when(b + 1 < NB)
        def _wait_pref():
            _drain(b + 1, 1 - p)

    @pl.loop(0, NB, step=2)
    def _batch(b):
        _do_batch(b, 0)

        @pl.when(b + 1 < NB)
        def _odd():
            _do_batch(b + 1, 1)

    plsc.subcore_barrier()

    for k in range(ROWS_PER_TILE // G):
        r0 = sid * ROWS_PER_TILE + k * G
        pltpu.sync_copy(acc.at[pl.ds(r0, G)],
                        out_hbm.at[pl.ds(cid * NP + r0, G)])


_sc_spmm = functools.partial(
    pl.kernel,
    out_type=jax.ShapeDtypeStruct((2 * NP, D), jnp.float32),
    mesh=plsc.VectorSubcoreMesh(core_axis_name="c", subcore_axis_name="s"),
    scratch_types=[
        pltpu.VMEM((2, G), jnp.int32),
        pltpu.VMEM((2, G), jnp.int32),
        pltpu.VMEM((2, G), jnp.int32),
        pltpu.VMEM((2, G), jnp.int32),
        pltpu.VMEM((2, G), jnp.int32),
        pltpu.VMEM((2, G), jnp.int32),
        pltpu.VMEM((G, D), jnp.float32),
        pltpu.VMEM((G, D), jnp.float32),
        pltpu.VMEM_SHARED((NP, D), jnp.float32),
        pltpu.SemaphoreType.DMA,
        pltpu.SemaphoreType.DMA,
    ],
    compiler_params=pltpu.CompilerParams(needs_layout_passes=False),
)(_sc_spmm_body)


_NB_ROWS = 2000
_RST = _NB_ROWS // Q


def _epilogue_body(with_fc, part_ref, pst_ref, x_ref, nt_ref, wt_ref, b_ref,
                   g_ref, bb_ref, pmat_ref, msel_ref, fcw_ref, fcb_ref,
                   out_ref):
    w = part_ref[0] + part_ref[1]
    stb = pst_ref[0] + pst_ref[1]
    u = jnp.dot(pmat_ref[...], stb, preferred_element_type=jnp.float32)
    s = jnp.sum(u * msel_ref[0], axis=-1, keepdims=True)
    t = jnp.sum(u * msel_ref[1], axis=-1, keepdims=True)
    aggr = w - s * x_ref[...] + t
    o0 = jnp.dot(aggr, wt_ref[0], preferred_element_type=jnp.float32) + b_ref[0:1, :]
    o1 = jnp.dot(aggr, wt_ref[1], preferred_element_type=jnp.float32) + b_ref[1:2, :]
    nt = nt_ref[...]
    h = jnp.where(nt == 0, o0, o1)
    mu = jnp.mean(h, axis=-1, keepdims=True)
    var = jnp.mean((h - mu) * (h - mu), axis=-1, keepdims=True)
    h = (h - mu) * lax.rsqrt(var + 1e-5) * g_ref[...] + bb_ref[...]
    h = jnp.maximum(h, 0.0)
    if with_fc:
        h = jnp.dot(h, fcw_ref[...], preferred_element_type=jnp.float32) + fcb_ref[...]
    out_ref[...] = h


def _epilogue(parts, x, nt2, wt, b, g, bb, pmat, msel, fcw, fcb, with_fc):
    grid = N // _NB_ROWS
    st0 = N // _RST
    return pl.pallas_call(
        functools.partial(_epilogue_body, with_fc),
        grid=(grid,),
        in_specs=[
            pl.BlockSpec((2, _NB_ROWS, D), lambda i: (0, i, 0)),
            pl.BlockSpec((2, _RST, D), lambda i: (0, st0 + i, 0)),
            pl.BlockSpec((_NB_ROWS, D), lambda i: (i, 0)),
            pl.BlockSpec((_NB_ROWS, 1), lambda i: (i, 0)),
            pl.BlockSpec((NT, D, D), lambda i: (0, 0, 0)),
            pl.BlockSpec((NT, D), lambda i: (0, 0)),
            pl.BlockSpec((1, D), lambda i: (0, 0)),
            pl.BlockSpec((1, D), lambda i: (0, 0)),
            pl.BlockSpec((_NB_ROWS, _RST), lambda i: (0, 0)),
            pl.BlockSpec((2, _NB_ROWS, D), lambda i: (0, 0, 0)),
            pl.BlockSpec((D, D), lambda i: (0, 0)),
            pl.BlockSpec((1, D), lambda i: (0, 0)),
        ],
        out_specs=pl.BlockSpec((_NB_ROWS, D), lambda i: (i, 0)),
        out_shape=jax.ShapeDtypeStruct((N, D), jnp.float32),
    )(parts, parts, x, nt2, wt, b, g, bb, pmat, msel, fcw, fcb)


def kernel(x, edge_index, edge_attr, node_type, edge_type,
           lin_W, lin_b, e0_W, e0_b, e1_W, e1_b, emb, ln_g, ln_b, fc_W, fc_b):
    wc_cols = []
    off_cols = []
    for l in range(L):
        wc_cols += [e0_W[l, 0], e1_W[l, 0], e1_W[l, 1]]
        off_cols += [emb[l] @ e0_W[l, 0] + e0_b[l, 0],
                     emb[l] @ e1_W[l, 0] + e1_b[l, 0],
                     emb[l] @ e1_W[l, 1] + e1_b[l, 1]]
    wc = jnp.pad(jnp.stack(wc_cols, axis=1), ((0, 0), (0, 2)))
    off = jnp.pad(jnp.stack(off_cols, axis=1), ((0, 0), (0, 2)))
    et2 = edge_type.reshape(E, 1)

    coef = _edge_coeffs(edge_attr, et2, wc, off)

    pad_w = ((0, 0), (0, EPW_P - EPW))
    src32 = jnp.pad(edge_index[0].reshape(NW, EPW), pad_w).reshape(NW, NB, G)
    dst32 = jnp.pad(edge_index[1].reshape(NW, EPW), pad_w).reshape(NW, NB, G)
    strow32 = N + dst32 // Q
    col32 = (dst32 % Q) * 2
    acs = [lax.bitcast_convert_type(
               jnp.pad(coef[:, j].reshape(NW, EPW), pad_w), jnp.int32
           ).reshape(NW, NB, G) for j in range(2 * L)]
    metas = [jnp.stack([src32, dst32, strow32, col32,
                        acs[2 * l], acs[2 * l + 1]],
                       axis=2).reshape(NW * NB * 6, G) for l in range(L)]

    nt2 = node_type.reshape(N, 1)
    jr = jnp.arange(_NB_ROWS)
    pmat = (jr[:, None] // Q == jnp.arange(_RST)[None, :]).astype(jnp.float32)
    lanes = jnp.arange(D)[None, :]
    msel = jnp.stack([(lanes == 2 * (jr[:, None] % Q)).astype(jnp.float32),
                      (lanes == 2 * (jr[:, None] % Q) + 1).astype(jnp.float32)])

    h = x
    for l in range(L):
        parts = _sc_spmm(h, metas[l])
        parts = parts.reshape(2, NP, D)
        wt = jnp.swapaxes(lin_W[l], -1, -2)
        h = _epilogue(parts, h, nt2, wt, lin_b[l],
                      ln_g[l].reshape(1, D), ln_b[l].reshape(1, D),
                      pmat, msel, fc_W.T, fc_b.reshape(1, D),
                      with_fc=(l == L - 1))
    return h


# speedup vs baseline: 3.1354x; 1.0163x over previous
import functools

import jax
import jax.numpy as jnp
from jax import lax
from jax.experimental import pallas as pl
from jax.experimental.pallas import tpu as pltpu
from jax.experimental.pallas import tpu_sc as plsc

N = 10000
E = 320000
D = 128
ED = 16
NT = 2
ET = 2
L = 2

NW = 32
G = 128
EPW = E // NW
NB = (EPW + G - 1) // G
EPW_P = NB * G
Q = 50
NST = N // Q
NP = 10240
ROWS_PER_TILE = NP // 16


def _edge_coeff_body(ea_ref, et_ref, wc_ref, off_ref, out_ref):
    ea = ea_ref[...]
    et = et_ref[...]
    raw = jnp.dot(ea, wc_ref[...], preferred_element_type=jnp.float32)
    off = jnp.where(et == 0, off_ref[0:1, :], off_ref[1:2, :])
    raw = raw + off
    d = ea[:, ED - 2:ED - 1]
    p = ea[:, ED - 1:ED]
    sign = d * 2.0 - 1.0
    speed_local = p * jnp.where(d > 0.0, d, 1.0)
    m0 = (et == 0).astype(jnp.float32)
    m1 = (et == 1).astype(jnp.float32)

    def softplus(v):
        return jnp.maximum(v, 0.0) + jnp.log1p(jnp.exp(-jnp.abs(v)))

    cols = []
    for l in range(L):
        r0 = raw[:, 3 * l + 0:3 * l + 1]
        r1a = raw[:, 3 * l + 1:3 * l + 2]
        r1b = raw[:, 3 * l + 2:3 * l + 3]
        gain = m0 * softplus(r0) + m1 * softplus(r1a) * speed_local
        bias = m1 * r1b * speed_local
        cols.append(sign * gain)
        cols.append(sign * bias)
    out_ref[...] = jnp.concatenate(cols, axis=1)


_EB = 8000


def _edge_coeffs(edge_attr, edge_type2, wc, off):
    grid = E // _EB
    return pl.pallas_call(
        _edge_coeff_body,
        grid=(grid,),
        in_specs=[
            pl.BlockSpec((_EB, ED), lambda i: (i, 0)),
            pl.BlockSpec((_EB, 1), lambda i: (i, 0)),
            pl.BlockSpec((ED, 8), lambda i: (0, 0)),
            pl.BlockSpec((ET, 8), lambda i: (0, 0)),
        ],
        out_specs=pl.BlockSpec((_EB, 2 * L), lambda i: (i, 0)),
        out_shape=jax.ShapeDtypeStruct((E, 2 * L), jnp.float32),
    )(edge_attr, edge_type2, wc, off)


def _sc_spmm_body(x_hbm, src_hbm, dst_hbm, str_hbm, col_hbm, a_hbm, c_hbm,
                  out_hbm, srcb, dstb, strb, colb, ab, cb,
                  rows, strows, acc, sem, msem):
    cid = lax.axis_index("c")
    sid = lax.axis_index("s")
    wid = cid * 16 + sid

    zero16 = jnp.zeros((16,), jnp.float32)

    @pl.loop(0, G)
    def _zero_rows(i):
        for j in range(D // 16):
            rows[i, pl.ds(j * 16, 16)] = zero16
            strows[i, pl.ds(j * 16, 16)] = zero16

    for k in range(ROWS_PER_TILE // G):
        pltpu.sync_copy(rows, acc.at[pl.ds(sid * ROWS_PER_TILE + k * G, G)])
    plsc.subcore_barrier()

    lane = lax.iota(jnp.int32, 16)

    pairs = [(src_hbm, srcb), (dst_hbm, dstb), (str_hbm, strb),
             (col_hbm, colb), (a_hbm, ab), (c_hbm, cb)]
    mbase = wid * NB

    def _fetch(b, p):
        for hbm, buf in pairs:
            pltpu.async_copy(hbm.at[mbase + b], buf.at[p], msem)

    def _drain(b, p):
        for hbm, buf in pairs:
            pltpu.make_async_copy(hbm.at[mbase + b], buf.at[p], msem).wait()

    _fetch(0, 0)
    _drain(0, 0)

    def _do_batch(b, p):
        @pl.when(b + 1 < NB)
        def _pref():
            _fetch(b + 1, 1 - p)

        pltpu.async_copy(x_hbm.at[srcb.at[p]], rows, sem).wait()

        @pl.loop(0, G // 16)
        def _grp(gg):
            sl16 = pl.ds(gg * 16, 16)
            a16 = ab[p, sl16]
            c16 = cb[p, sl16]
            col16 = colb[p, sl16]
            g16 = lane + gg * 16
            plsc.store_scatter(strows, [g16, col16], a16)
            plsc.store_scatter(strows, [g16, col16 + 1], c16)
            for j in range(16):
                g = gg * 16 + j
                av = jnp.full((16,), a16[j], jnp.float32)
                for dcol in range(D // 16):
                    sl = pl.ds(dcol * 16, 16)
                    rows[g, sl] = rows[g, sl] * av

        pltpu.sync_copy(rows, acc.at[dstb.at[p]], add=True)
        pltpu.sync_copy(strows, acc.at[strb.at[p]], add=True)

        @pl.loop(0, G // 16)
        def _unzero(gg):
            col16 = colb[p, pl.ds(gg * 16, 16)]
            g16 = lane + gg * 16
            plsc.store_scatter(strows, [g16, col16], zero16)
            plsc.store_scatter(strows, [g16, col16 + 1], zero16)

        @pl.when(b + 1 < NB)
        def _wait_pref():
            _drain(b + 1, 1 - p)

    @pl.loop(0, NB, step=2)
    def _batch(b):
        _do_batch(b, 0)

        @pl.when(b + 1 < NB)
        def _odd():
            _do_batch(b + 1, 1)

    plsc.subcore_barrier()

    for k in range(ROWS_PER_TILE // G):
        r0 = sid * ROWS_PER_TILE + k * G
        pltpu.sync_copy(acc.at[pl.ds(r0, G)],
                        out_hbm.at[pl.ds(cid * NP + r0, G)])


_sc_spmm = functools.partial(
    pl.kernel,
    out_type=jax.ShapeDtypeStruct((2 * NP, D), jnp.float32),
    mesh=plsc.VectorSubcoreMesh(core_axis_name="c", subcore_axis_name="s"),
    scratch_types=[
        pltpu.VMEM((2, G), jnp.int32),
        pltpu.VMEM((2, G), jnp.int32),
        pltpu.VMEM((2, G), jnp.int32),
        pltpu.VMEM((2, G), jnp.int32),
        pltpu.VMEM((2, G), jnp.float32),
        pltpu.VMEM((2, G), jnp.float32),
        pltpu.VMEM((G, D), jnp.float32),
        pltpu.VMEM((G, D), jnp.float32),
        pltpu.VMEM_SHARED((NP, D), jnp.float32),
        pltpu.SemaphoreType.DMA,
        pltpu.SemaphoreType.DMA,
    ],
    compiler_params=pltpu.CompilerParams(needs_layout_passes=False),
)(_sc_spmm_body)


_NB_ROWS = 2000
_RST = _NB_ROWS // Q


def _epilogue_body(with_fc, part_ref, pst_ref, x_ref, nt_ref, wt_ref, b_ref,
                   g_ref, bb_ref, pmat_ref, msel_ref, fcw_ref, fcb_ref,
                   out_ref):
    w = part_ref[0] + part_ref[1]
    stb = pst_ref[0] + pst_ref[1]
    u = jnp.dot(pmat_ref[...], stb, preferred_element_type=jnp.float32)
    s = jnp.sum(u * msel_ref[0], axis=-1, keepdims=True)
    t = jnp.sum(u * msel_ref[1], axis=-1, keepdims=True)
    aggr = w - s * x_ref[...] + t
    o0 = jnp.dot(aggr, wt_ref[0], preferred_element_type=jnp.float32) + b_ref[0:1, :]
    o1 = jnp.dot(aggr, wt_ref[1], preferred_element_type=jnp.float32) + b_ref[1:2, :]
    nt = nt_ref[...]
    h = jnp.where(nt == 0, o0, o1)
    mu = jnp.mean(h, axis=-1, keepdims=True)
    var = jnp.mean((h - mu) * (h - mu), axis=-1, keepdims=True)
    h = (h - mu) * lax.rsqrt(var + 1e-5) * g_ref[...] + bb_ref[...]
    h = jnp.maximum(h, 0.0)
    if with_fc:
        h = jnp.dot(h, fcw_ref[...], preferred_element_type=jnp.float32) + fcb_ref[...]
    out_ref[...] = h


def _epilogue(parts, x, nt2, wt, b, g, bb, pmat, msel, fcw, fcb, with_fc):
    grid = N // _NB_ROWS
    st0 = N // _RST
    return pl.pallas_call(
        functools.partial(_epilogue_body, with_fc),
        grid=(grid,),
        in_specs=[
            pl.BlockSpec((2, _NB_ROWS, D), lambda i: (0, i, 0)),
            pl.BlockSpec((2, _RST, D), lambda i: (0, st0 + i, 0)),
            pl.BlockSpec((_NB_ROWS, D), lambda i: (i, 0)),
            pl.BlockSpec((_NB_ROWS, 1), lambda i: (i, 0)),
            pl.BlockSpec((NT, D, D), lambda i: (0, 0, 0)),
            pl.BlockSpec((NT, D), lambda i: (0, 0)),
            pl.BlockSpec((1, D), lambda i: (0, 0)),
            pl.BlockSpec((1, D), lambda i: (0, 0)),
            pl.BlockSpec((_NB_ROWS, _RST), lambda i: (0, 0)),
            pl.BlockSpec((2, _NB_ROWS, D), lambda i: (0, 0, 0)),
            pl.BlockSpec((D, D), lambda i: (0, 0)),
            pl.BlockSpec((1, D), lambda i: (0, 0)),
        ],
        out_specs=pl.BlockSpec((_NB_ROWS, D), lambda i: (i, 0)),
        out_shape=jax.ShapeDtypeStruct((N, D), jnp.float32),
    )(parts, parts, x, nt2, wt, b, g, bb, pmat, msel, fcw, fcb)


def kernel(x, edge_index, edge_attr, node_type, edge_type,
           lin_W, lin_b, e0_W, e0_b, e1_W, e1_b, emb, ln_g, ln_b, fc_W, fc_b):
    wc_cols = []
    off_cols = []
    for l in range(L):
        wc_cols += [e0_W[l, 0], e1_W[l, 0], e1_W[l, 1]]
        off_cols += [emb[l] @ e0_W[l, 0] + e0_b[l, 0],
                     emb[l] @ e1_W[l, 0] + e1_b[l, 0],
                     emb[l] @ e1_W[l, 1] + e1_b[l, 1]]
    wc = jnp.pad(jnp.stack(wc_cols, axis=1), ((0, 0), (0, 2)))
    off = jnp.pad(jnp.stack(off_cols, axis=1), ((0, 0), (0, 2)))
    et2 = edge_type.reshape(E, 1)

    coef = _edge_coeffs(edge_attr, et2, wc, off)

    pad_w = ((0, 0), (0, EPW_P - EPW))
    src32 = jnp.pad(edge_index[0].reshape(NW, EPW), pad_w).reshape(NW, NB, G)
    dst32 = jnp.pad(edge_index[1].reshape(NW, EPW), pad_w).reshape(NW, NB, G)
    strow32 = N + dst32 // Q
    col32 = (dst32 % Q) * 2
    acs = [jnp.pad(coef[:, j].reshape(NW, EPW),
                   pad_w).reshape(NW * NB, G) for j in range(2 * L)]
    src_f = src32.reshape(NW * NB, G)
    dst_f = dst32.reshape(NW * NB, G)
    str_f = strow32.reshape(NW * NB, G)
    col_f = col32.reshape(NW * NB, G)

    nt2 = node_type.reshape(N, 1)
    jr = jnp.arange(_NB_ROWS)
    pmat = (jr[:, None] // Q == jnp.arange(_RST)[None, :]).astype(jnp.float32)
    lanes = jnp.arange(D)[None, :]
    msel = jnp.stack([(lanes == 2 * (jr[:, None] % Q)).astype(jnp.float32),
                      (lanes == 2 * (jr[:, None] % Q) + 1).astype(jnp.float32)])

    h = x
    for l in range(L):
        parts = _sc_spmm(h, src_f, dst_f, str_f, col_f,
                         acs[2 * l], acs[2 * l + 1])
        parts = parts.reshape(2, NP, D)
        wt = jnp.swapaxes(lin_W[l], -1, -2)
        h = _epilogue(parts, h, nt2, wt, lin_b[l],
                      ln_g[l].reshape(1, D), ln_b[l].reshape(1, D),
                      pmat, msel, fc_W.T, fc_b.reshape(1, D),
                      with_fc=(l == L - 1))
    return h
